# Initial kernel scaffold; baseline (speedup 1.0000x reference)
import jax
import jax.numpy as jnp
from jax.experimental import pallas as pl


def kernel(boardInts, boardFeats, twEmb, trEmb, weatherEmb, terrainEmb, ln_g, ln_b, W, b):
    raise NotImplementedError("write your pallas kernel here")


# speedup vs baseline: 12.4300x; 12.4300x over previous
import functools

import jax
import jax.numpy as jnp
from jax import lax
from jax.experimental import pallas as pl
from jax.experimental.pallas import tpu as pltpu

_NEMB = 4
_NFEATS = 15
_NHIDDEN = 128
_NEWDIM = 3 * _NEMB + _NEMB + _NEMB + _NFEATS
_EPS = 1e-5


def _board_kernel(ints_ref, feats_ref, tw_ref, tr_ref, we_ref, te_ref,
                  g_ref, beta_ref, w_ref, b_ref, out_ref):
    R = ints_ref.shape[0]
    ints = ints_ref[...]
    feats = feats_ref[...]
    iota5 = lax.broadcasted_iota(jnp.int32, (R, 5), 1)

    tables = (tw_ref, tw_ref, tr_ref, we_ref, te_ref)
    parts = []
    for c, t_ref in enumerate(tables):
        oh = (ints[:, c:c + 1] == iota5).astype(jnp.float32)
        parts.append(jnp.dot(oh, t_ref[...],
                             preferred_element_type=jnp.float32))
    parts.append(feats)
    comb = jnp.concatenate(parts, axis=-1)

    mu = jnp.mean(comb, axis=-1, keepdims=True)
    var = jnp.mean((comb - mu) ** 2, axis=-1, keepdims=True)
    normed = (comb - mu) * lax.rsqrt(var + _EPS) * g_ref[...] + beta_ref[...]
    y = jnp.dot(normed, w_ref[...], preferred_element_type=jnp.float32)
    out_ref[...] = jnp.maximum(y + b_ref[...], 0.0)


@functools.partial(jax.jit, static_argnames=("block_r",))
def _run(boardInts, boardFeats, twEmb, trEmb, weatherEmb, terrainEmb,
         ln_g, ln_b, W, b, block_r=2048):
    B = boardInts.shape[0]
    grid = (B // block_r,)
    row_spec = lambda width: pl.BlockSpec((block_r, width), lambda i: (i, 0))
    full = lambda shape: pl.BlockSpec(shape, lambda i: (0,) * len(shape))
    return pl.pallas_call(
        _board_kernel,
        grid=grid,
        in_specs=[
            row_spec(5),
            row_spec(_NFEATS),
            full((5, _NEMB)),
            full((5, _NEMB)),
            full((5, _NEMB)),
            full((5, _NEMB)),
            full((_NEWDIM,)),
            full((_NEWDIM,)),
            full((_NEWDIM, _NHIDDEN)),
            full((_NHIDDEN,)),
        ],
        out_specs=row_spec(_NHIDDEN),
        out_shape=jax.ShapeDtypeStruct((B, _NHIDDEN), jnp.float32),
    )(boardInts, boardFeats, twEmb, trEmb, weatherEmb, terrainEmb,
      ln_g, ln_b, W, b)


def kernel(boardInts, boardFeats, twEmb, trEmb, weatherEmb, terrainEmb,
           ln_g, ln_b, W, b):
    return _run(boardInts, boardFeats, twEmb, trEmb, weatherEmb, terrainEmb,
                ln_g, ln_b, W, b)


# speedup vs baseline: 35.8961x; 2.8879x over previous
import functools

import jax
import jax.numpy as jnp
from jax import lax
from jax.experimental import pallas as pl

_NEMB = 4
_NFEATS = 15
_NHIDDEN = 128
_NEWDIM = 3 * _NEMB + _NEMB + _NEMB + _NFEATS
_NTAB = 5
_EPS = 1e-5


def _board_kernel(intsT_ref, featsT_ref, gmap_ref, gvec_ref,
                  betavec_ref, w_ref, b2_ref, out_ref):
    intsT = intsT_ref[...]
    featsT = featsT_ref[...]

    rep = jnp.concatenate([intsT] * _NTAB, axis=0)
    val = lax.broadcasted_iota(jnp.int32, (5 * _NTAB, 1), 0) // _NTAB
    ohT = (rep == val).astype(jnp.float32)

    embT = jnp.dot(gmap_ref[...], ohT,
                   preferred_element_type=jnp.float32)
    combT = jnp.concatenate([embT, featsT], axis=0)

    mu = jnp.mean(combT, axis=0, keepdims=True)
    var = jnp.mean((combT - mu) ** 2, axis=0, keepdims=True)
    normT = ((combT - mu) * lax.rsqrt(var + _EPS) * gvec_ref[...]
             + betavec_ref[...])

    y = lax.dot_general(normT, w_ref[...],
                        dimension_numbers=(((0,), (0,)), ((), ())),
                        preferred_element_type=jnp.float32)
    out_ref[...] = jnp.maximum(y + b2_ref[...], 0.0)


@functools.partial(jax.jit, static_argnames=("block_r",))
def _run(boardInts, boardFeats, twEmb, trEmb, weatherEmb, terrainEmb,
         ln_g, ln_b, W, b, block_r=2048):
    B = boardInts.shape[0]
    intsT = boardInts.T
    featsT = boardFeats.T

    tables = jnp.stack([twEmb, twEmb, trEmb, weatherEmb, terrainEmb])
    t_ckv = jnp.transpose(tables, (0, 2, 1))
    gmap = (t_ckv[:, :, :, None] * jnp.eye(_NTAB, dtype=jnp.float32)[:, None, None, :]
            ).reshape(4 * _NTAB, 5 * _NTAB)

    gvec = ln_g.reshape(_NEWDIM, 1)
    betavec = ln_b.reshape(_NEWDIM, 1)
    b2 = b.reshape(1, _NHIDDEN)

    grid = (B // block_r,)
    full = lambda shape: pl.BlockSpec(shape, lambda i: (0,) * len(shape))
    return pl.pallas_call(
        _board_kernel,
        grid=grid,
        in_specs=[
            pl.BlockSpec((5, block_r), lambda i: (0, i)),
            pl.BlockSpec((_NFEATS, block_r), lambda i: (0, i)),
            full((4 * _NTAB, 5 * _NTAB)),
            full((_NEWDIM, 1)),
            full((_NEWDIM, 1)),
            full((_NEWDIM, _NHIDDEN)),
            full((1, _NHIDDEN)),
        ],
        out_specs=pl.BlockSpec((block_r, _NHIDDEN), lambda i: (i, 0)),
        out_shape=jax.ShapeDtypeStruct((B, _NHIDDEN), jnp.float32),
    )(intsT, featsT, gmap, gvec, betavec, W, b2)


def kernel(boardInts, boardFeats, twEmb, trEmb, weatherEmb, terrainEmb,
           ln_g, ln_b, W, b):
    return _run(boardInts, boardFeats, twEmb, trEmb, weatherEmb, terrainEmb,
                ln_g, ln_b, W, b)


# speedup vs baseline: 40.5731x; 1.1303x over previous
import functools

import jax
import jax.numpy as jnp
from jax import lax
from jax.experimental import pallas as pl

_NEMB = 4
_NFEATS = 15
_NHIDDEN = 128
_NEWDIM = 3 * _NEMB + _NEMB + _NEMB + _NFEATS
_NTAB = 5
_EPS = 1e-5


def _board_kernel(intsT_ref, featsT_ref, gmap_ref, gvec_ref,
                  betavec_ref, w_ref, b2_ref, out_ref):
    intsT = intsT_ref[...]
    featsT = featsT_ref[...]

    rep = jnp.concatenate([intsT] * _NTAB, axis=0)
    val = lax.broadcasted_iota(jnp.int32, (5 * _NTAB, 1), 0) // _NTAB
    ohT = (rep == val).astype(jnp.float32)

    embT = jnp.dot(gmap_ref[...], ohT,
                   preferred_element_type=jnp.float32)
    combT = jnp.concatenate([embT, featsT], axis=0)

    mu = jnp.mean(combT, axis=0, keepdims=True)
    var = jnp.mean((combT - mu) ** 2, axis=0, keepdims=True)
    normT = ((combT - mu) * lax.rsqrt(var + _EPS) * gvec_ref[...]
             + betavec_ref[...])

    y = lax.dot_general(normT, w_ref[...],
                        dimension_numbers=(((0,), (0,)), ((), ())),
                        preferred_element_type=jnp.float32)
    out_ref[...] = jnp.maximum(y + b2_ref[...], 0.0)


@functools.partial(jax.jit, static_argnames=("block_r",))
def _run(boardInts, boardFeats, twEmb, trEmb, weatherEmb, terrainEmb,
         ln_g, ln_b, W, b, block_r=4096):
    B = boardInts.shape[0]
    intsT = boardInts.T
    featsT = boardFeats.T

    tables = jnp.stack([twEmb, twEmb, trEmb, weatherEmb, terrainEmb])
    t_ckv = jnp.transpose(tables, (0, 2, 1))
    gmap = (t_ckv[:, :, :, None] * jnp.eye(_NTAB, dtype=jnp.float32)[:, None, None, :]
            ).reshape(4 * _NTAB, 5 * _NTAB)

    gvec = ln_g.reshape(_NEWDIM, 1)
    betavec = ln_b.reshape(_NEWDIM, 1)
    b2 = b.reshape(1, _NHIDDEN)

    grid = (B // block_r,)
    full = lambda shape: pl.BlockSpec(shape, lambda i: (0,) * len(shape))
    return pl.pallas_call(
        _board_kernel,
        grid=grid,
        in_specs=[
            pl.BlockSpec((5, block_r), lambda i: (0, i)),
            pl.BlockSpec((_NFEATS, block_r), lambda i: (0, i)),
            full((4 * _NTAB, 5 * _NTAB)),
            full((_NEWDIM, 1)),
            full((_NEWDIM, 1)),
            full((_NEWDIM, _NHIDDEN)),
            full((1, _NHIDDEN)),
        ],
        out_specs=pl.BlockSpec((block_r, _NHIDDEN), lambda i: (i, 0)),
        out_shape=jax.ShapeDtypeStruct((B, _NHIDDEN), jnp.float32),
    )(intsT, featsT, gmap, gvec, betavec, W, b2)


def kernel(boardInts, boardFeats, twEmb, trEmb, weatherEmb, terrainEmb,
           ln_g, ln_b, W, b):
    return _run(boardInts, boardFeats, twEmb, trEmb, weatherEmb, terrainEmb,
                ln_g, ln_b, W, b)


# speedup vs baseline: 42.6308x; 1.0507x over previous
import functools

import jax
import jax.numpy as jnp
from jax import lax
from jax.experimental import pallas as pl

_NEMB = 4
_NFEATS = 15
_NHIDDEN = 128
_NEWDIM = 3 * _NEMB + _NEMB + _NEMB + _NFEATS
_NTAB = 5
_EPS = 1e-5


def _board_kernel(intsT_ref, featsT_ref, gmap_ref, gvec_ref,
                  betavec_ref, w_ref, b2_ref, out_ref):
    intsT = intsT_ref[...]
    featsT = featsT_ref[...]

    rep = jnp.concatenate([intsT] * _NTAB, axis=0)
    val = lax.broadcasted_iota(jnp.int32, (5 * _NTAB, 1), 0) // _NTAB
    ohT = (rep == val).astype(jnp.float32)

    embT = jnp.dot(gmap_ref[...], ohT,
                   preferred_element_type=jnp.float32)
    combT = jnp.concatenate([embT, featsT], axis=0)

    mu = jnp.mean(combT, axis=0, keepdims=True)
    var = jnp.mean((combT - mu) ** 2, axis=0, keepdims=True)
    normT = ((combT - mu) * lax.rsqrt(var + _EPS) * gvec_ref[...]
             + betavec_ref[...])

    y = lax.dot_general(normT, w_ref[...],
                        dimension_numbers=(((0,), (0,)), ((), ())),
                        preferred_element_type=jnp.float32)
    out_ref[...] = jnp.maximum(y + b2_ref[...], 0.0)


@functools.partial(jax.jit, static_argnames=("block_r",))
def _run(boardInts, boardFeats, twEmb, trEmb, weatherEmb, terrainEmb,
         ln_g, ln_b, W, b, block_r=8192):
    B = boardInts.shape[0]
    intsT = boardInts.T
    featsT = boardFeats.T

    tables = jnp.stack([twEmb, twEmb, trEmb, weatherEmb, terrainEmb])
    t_ckv = jnp.transpose(tables, (0, 2, 1))
    gmap = (t_ckv[:, :, :, None] * jnp.eye(_NTAB, dtype=jnp.float32)[:, None, None, :]
            ).reshape(4 * _NTAB, 5 * _NTAB)

    gvec = ln_g.reshape(_NEWDIM, 1)
    betavec = ln_b.reshape(_NEWDIM, 1)
    b2 = b.reshape(1, _NHIDDEN)

    grid = (B // block_r,)
    full = lambda shape: pl.BlockSpec(shape, lambda i: (0,) * len(shape))
    return pl.pallas_call(
        _board_kernel,
        grid=grid,
        in_specs=[
            pl.BlockSpec((5, block_r), lambda i: (0, i)),
            pl.BlockSpec((_NFEATS, block_r), lambda i: (0, i)),
            full((4 * _NTAB, 5 * _NTAB)),
            full((_NEWDIM, 1)),
            full((_NEWDIM, 1)),
            full((_NEWDIM, _NHIDDEN)),
            full((1, _NHIDDEN)),
        ],
        out_specs=pl.BlockSpec((block_r, _NHIDDEN), lambda i: (i, 0)),
        out_shape=jax.ShapeDtypeStruct((B, _NHIDDEN), jnp.float32),
    )(intsT, featsT, gmap, gvec, betavec, W, b2)


def kernel(boardInts, boardFeats, twEmb, trEmb, weatherEmb, terrainEmb,
           ln_g, ln_b, W, b):
    return _run(boardInts, boardFeats, twEmb, trEmb, weatherEmb, terrainEmb,
                ln_g, ln_b, W, b)
